# baseline (speedup 1.0000x reference)
import jax
import jax.numpy as jnp
from jax import lax
from jax.experimental import pallas as pl
from jax.experimental.pallas import tpu as pltpu
from jax.experimental.pallas import tpu_sc as plsc

_NC = 2
_NS = 16
_NW = _NC * _NS
_CHUNK = 128
_EB = 4096


def _sc_gather(table, idx3d, e_pad):
    n, d = table.shape
    per_w = e_pad // _NW
    k = per_w // _CHUNK
    ngrp = 1 if k <= 5 else 2
    kg = k // ngrp
    grp = per_w // ngrp
    mesh = plsc.VectorSubcoreMesh(core_axis_name="c", subcore_axis_name="s")

    def body(tab_hbm, idx_hbm, out_hbm, idx_v, rows_v, sem):
        c = lax.axis_index("c")
        s = lax.axis_index("s")
        wid = s * _NC + c
        pltpu.sync_copy(idx_hbm.at[wid], idx_v)
        for hf in range(ngrp):
            cps = [
                pltpu.async_copy(
                    tab_hbm.at[idx_v.at[hf * kg + j]],
                    rows_v.at[pl.ds(j * _CHUNK, _CHUNK)],
                    sem,
                )
                for j in range(kg)
            ]
            for cp in cps:
                cp.wait()
            pltpu.sync_copy(
                rows_v, out_hbm.at[pl.ds(wid * per_w + hf * grp, grp)])

    fn = pl.kernel(
        body,
        out_type=jax.ShapeDtypeStruct((e_pad, d), jnp.float32),
        mesh=mesh,
        scratch_types=[
            pltpu.VMEM((k, _CHUNK), jnp.int32),
            pltpu.VMEM((grp, d), jnp.float32),
            pltpu.SemaphoreType.DMA,
        ],
        compiler_params=pltpu.CompilerParams(use_tc_tiling_on_sc=False),
    )
    return fn(table, idx3d)


def _sc_scatter_add(msg, idx3d, zeros_init, n_sc):
    e_pad, d = msg.shape
    per_w = e_pad // _NW
    k = per_w // _CHUNK
    ngrp = 1 if k <= 5 else 2
    kg = k // ngrp
    grp = per_w // ngrp
    rows_per_s = n_sc // _NS
    mesh = plsc.VectorSubcoreMesh(core_axis_name="c", subcore_axis_name="s")

    def body(msg_hbm, idx_hbm, zero_hbm, out_hbm, idx_v, msg_v, acc_sh):
        c = lax.axis_index("c")
        s = lax.axis_index("s")
        wid = s * _NC + c
        pltpu.sync_copy(
            zero_hbm.at[pl.ds(s * rows_per_s, rows_per_s)],
            acc_sh.at[pl.ds(s * rows_per_s, rows_per_s)],
        )
        plsc.subcore_barrier()
        pltpu.sync_copy(idx_hbm.at[wid], idx_v)
        for hf in range(ngrp):
            pltpu.sync_copy(
                msg_hbm.at[pl.ds(wid * per_w + hf * grp, grp)], msg_v)
            for j in range(kg):
                pltpu.sync_copy(
                    msg_v.at[pl.ds(j * _CHUNK, _CHUNK)],
                    acc_sh.at[idx_v.at[hf * kg + j]],
                    add=True,
                )
        plsc.subcore_barrier()
        pltpu.sync_copy(
            acc_sh.at[pl.ds(s * rows_per_s, rows_per_s)],
            out_hbm.at[pl.ds(c * n_sc + s * rows_per_s, rows_per_s)],
        )

    fn = pl.kernel(
        body,
        out_type=jax.ShapeDtypeStruct((2 * n_sc, d), jnp.float32),
        mesh=mesh,
        scratch_types=[
            pltpu.VMEM((k, _CHUNK), jnp.int32),
            pltpu.VMEM((grp, d), jnp.float32),
            pltpu.VMEM_SHARED((n_sc, d), jnp.float32),
        ],
        compiler_params=pltpu.CompilerParams(use_tc_tiling_on_sc=False),
    )
    return fn(msg, idx3d, zeros_init)


def _edge_messages(ea_ext, xj, We1e, We2e, R128):
    e_pad, bfe = ea_ext.shape
    kc = We2e.shape[1]
    nblocks = kc // 128
    h_d = We1e.shape[1]

    def body(ea_ref, xj_ref, We1_ref, We2_ref, R_ref, out_ref):
        h = jnp.maximum(
            jnp.dot(ea_ref[...], We1_ref[...],
                    preferred_element_type=jnp.float32), 0.0)
        hb = h.astype(jnp.bfloat16)
        xb = xj_ref[...].astype(jnp.bfloat16)
        ngr = 8
        gw = kc // ngr
        acc = None
        for gr in range(ngr):
            gsl = slice(gr * gw, (gr + 1) * gw)
            Wt = jnp.dot(hb, We2_ref[:, gsl],
                         preferred_element_type=jnp.float32)
            XR = jnp.dot(xb, R_ref[:, gsl],
                         preferred_element_type=jnp.float32)
            for j in range(gw // 128):
                sl = slice(j * 128, (j + 1) * 128)
                t = Wt[:, sl] * XR[:, sl]
                acc = t if acc is None else acc + t
        out_ref[...] = acc

    return pl.pallas_call(
        body,
        grid=(e_pad // _EB,),
        in_specs=[
            pl.BlockSpec((_EB, bfe), lambda i: (i, 0)),
            pl.BlockSpec((_EB, 128), lambda i: (i, 0)),
            pl.BlockSpec((bfe, h_d), lambda i: (0, 0)),
            pl.BlockSpec((h_d, kc), lambda i: (0, 0)),
            pl.BlockSpec((128, kc), lambda i: (0, 0)),
        ],
        out_specs=pl.BlockSpec((_EB, 128), lambda i: (i, 0)),
        out_shape=jax.ShapeDtypeStruct((e_pad, 128), jnp.float32),
    )(ea_ext, xj, We1e, We2e, R128)


def _node_update(partsA, partsB, x, root, bias, g, b, n, n_sc):
    def body(pa_ref, pb_ref, x_ref, root_ref, bias_ref, g_ref, b_ref,
             out_ref):
        s128 = (pa_ref[0:n, :] + pa_ref[n_sc:n_sc + n, :]
                + pb_ref[0:n, :] + pb_ref[n_sc:n_sc + n, :])
        agg = (s128[:, 0:64] + s128[:, 64:128]
               + jnp.dot(x_ref[:, 0:64], root_ref[...],
                         preferred_element_type=jnp.float32)
               + bias_ref[...])
        m = jnp.mean(agg, axis=-1, keepdims=True)
        cen = agg - m
        v = jnp.mean(cen * cen, axis=-1, keepdims=True)
        hout = jnp.maximum(
            cen * lax.rsqrt(v + 1e-5) * g_ref[...] + b_ref[...], 0.0)
        out_ref[...] = jnp.concatenate(
            [hout, jnp.zeros_like(hout)], axis=1)

    return pl.pallas_call(
        body,
        out_shape=jax.ShapeDtypeStruct((n, 128), jnp.float32),
    )(partsA, partsB, x, root, bias, g, b)


def _pool_head(h, batch_col, fc1_W, fc1_b, fc2_W, fc2_b, gpad):
    n = h.shape[0]
    out_d = fc2_W.shape[1]

    def body(h_ref, batch_ref, w1_ref, b1_ref, w2_ref, b2_ref, out_ref):
        gids = lax.broadcasted_iota(jnp.int32, (1, gpad), 1)
        oh = (batch_ref[...] == gids).astype(jnp.float32)
        dn = (((0,), (0,)), ((), ()))
        sums = lax.dot_general(oh, h_ref[:, 0:64], dn,
                               preferred_element_type=jnp.float32)
        ones = jnp.ones((n, 1), jnp.float32)
        cnts = lax.dot_general(oh, ones, dn,
                               preferred_element_type=jnp.float32)
        hg = sums / jnp.maximum(cnts, 1.0)
        h2 = jnp.maximum(
            jnp.dot(hg, w1_ref[...], preferred_element_type=jnp.float32)
            + b1_ref[...], 0.0)
        out_ref[...] = (
            jnp.dot(h2, w2_ref[...], preferred_element_type=jnp.float32)
            + b2_ref[...])

    return pl.pallas_call(
        body,
        out_shape=jax.ShapeDtypeStruct((gpad, out_d), jnp.float32),
    )(h, batch_col, fc1_W, fc1_b, fc2_W, fc2_b)


def kernel(x, edge_index, edge_attr, batch,
           We1_1, be1_1, We2_1, be2_1, root1, bias1, g1, b1,
           We1_2, be1_2, We2_2, be2_2, root2, bias2, g2, b2,
           We1_3, be1_3, We2_3, be2_3, root3, bias3, g3, b3,
           fc1_W, fc1_b, fc2_W, fc2_b):
    n, nf = x.shape
    e = edge_index.shape[1]
    bf = edge_attr.shape[1]
    num_graphs = 50

    align = _NW * _CHUNK
    e_pad = -(-e // align) * align
    n_sc = -(-(n + 1) // (_NS * 8)) * (_NS * 8)

    pad_e = e_pad - e
    src = jnp.concatenate([edge_index[0], jnp.zeros((pad_e,), jnp.int32)])
    dump = n + (jnp.arange(pad_e, dtype=jnp.int32) % (n_sc - n))
    dst = jnp.concatenate([edge_index[1], dump])
    e_half = e_pad // 2
    kk = e_half // (_NW * _CHUNK)
    src0 = src[:e_half].reshape(_NW, kk, _CHUNK)
    src1 = src[e_half:].reshape(_NW, kk, _CHUNK)
    dst0 = dst[:e_half].reshape(_NW, kk, _CHUNK)
    dst1 = dst[e_half:].reshape(_NW, kk, _CHUNK)
    ea_ext = jnp.concatenate(
        [edge_attr, jnp.ones((e, 1), jnp.float32)], axis=1)
    ea_ext = jnp.concatenate(
        [ea_ext, jnp.zeros((pad_e, bf + 1), jnp.float32)], axis=0)
    ea0 = ea_ext[:e_pad // 2]
    ea1 = ea_ext[e_pad // 2:]
    zeros_init = jnp.zeros((n_sc, 128), jnp.float32)
    R = jnp.kron(jnp.eye(64, dtype=jnp.float32),
                 jnp.ones((1, 64), jnp.float32))
    R128 = jnp.concatenate(
        [R, jnp.zeros((64, R.shape[1]), jnp.float32)],
        axis=0).astype(jnp.bfloat16)

    layers = [
        (We1_1, be1_1, We2_1, be2_1, root1, bias1, g1, b1),
        (We1_2, be1_2, We2_2, be2_2, root2, bias2, g2, b2),
        (We1_3, be1_3, We2_3, be2_3, root3, bias3, g3, b3),
    ]

    h = jnp.concatenate([x, jnp.zeros((n, 64), jnp.float32)], axis=1)
    for We1, be1, We2, be2, root, bias, g, b in layers:
        top = jnp.concatenate([We1, jnp.zeros((bf, 1), jnp.float32)], axis=1)
        bot = jnp.concatenate(
            [be1[None, :], jnp.ones((1, 1), jnp.float32)], axis=1)
        We1e = jnp.concatenate([top, bot], axis=0)
        We2e = jnp.concatenate(
            [We2, be2[None, :]], axis=0).astype(jnp.bfloat16)

        xj0 = _sc_gather(h, src0, e_half)
        msg0 = _edge_messages(ea0, xj0, We1e, We2e, R128)
        xj1 = _sc_gather(h, src1, e_half)
        partsA = _sc_scatter_add(msg0, dst0, zeros_init, n_sc)
        msg1 = _edge_messages(ea1, xj1, We1e, We2e, R128)
        partsB = _sc_scatter_add(msg1, dst1, zeros_init, n_sc)
        h = _node_update(partsA, partsB, h, root, bias.reshape(1, 64),
                         g.reshape(1, 64), b.reshape(1, 64), n, n_sc)

    out = _pool_head(h, batch.reshape(n, 1).astype(jnp.int32),
                     fc1_W, fc1_b.reshape(1, -1), fc2_W, fc2_b.reshape(1, -1),
                     gpad=64)
    return out[:num_graphs]


# speedup vs baseline: 1.0193x; 1.0193x over previous
import jax
import jax.numpy as jnp
from jax import lax
from jax.experimental import pallas as pl
from jax.experimental.pallas import tpu as pltpu
from jax.experimental.pallas import tpu_sc as plsc

_NC = 2
_NS = 16
_NW = _NC * _NS
_CHUNK = 128
_EB = 2048


def _sc_gather(table, idx3d, e_pad):
    n, d = table.shape
    per_w = e_pad // _NW
    k = per_w // _CHUNK
    ngrp = 1 if k <= 5 else 2
    kg = k // ngrp
    grp = per_w // ngrp
    mesh = plsc.VectorSubcoreMesh(core_axis_name="c", subcore_axis_name="s")

    def body(tab_hbm, idx_hbm, out_hbm, idx_v, rows_v, sem):
        c = lax.axis_index("c")
        s = lax.axis_index("s")
        wid = s * _NC + c
        pltpu.sync_copy(idx_hbm.at[wid], idx_v)
        for hf in range(ngrp):
            cps = [
                pltpu.async_copy(
                    tab_hbm.at[idx_v.at[hf * kg + j]],
                    rows_v.at[pl.ds(j * _CHUNK, _CHUNK)],
                    sem,
                )
                for j in range(kg)
            ]
            for cp in cps:
                cp.wait()
            pltpu.sync_copy(
                rows_v, out_hbm.at[pl.ds(wid * per_w + hf * grp, grp)])

    fn = pl.kernel(
        body,
        out_type=jax.ShapeDtypeStruct((e_pad, d), jnp.float32),
        mesh=mesh,
        scratch_types=[
            pltpu.VMEM((k, _CHUNK), jnp.int32),
            pltpu.VMEM((grp, d), jnp.float32),
            pltpu.SemaphoreType.DMA,
        ],
        compiler_params=pltpu.CompilerParams(use_tc_tiling_on_sc=False),
    )
    return fn(table, idx3d)


def _sc_scatter_add(msg, idx3d, zeros_init, n_sc):
    e_pad, d = msg.shape
    per_w = e_pad // _NW
    k = per_w // _CHUNK
    ngrp = 1 if k <= 5 else 2
    kg = k // ngrp
    grp = per_w // ngrp
    rows_per_s = n_sc // _NS
    mesh = plsc.VectorSubcoreMesh(core_axis_name="c", subcore_axis_name="s")

    def body(msg_hbm, idx_hbm, zero_hbm, out_hbm, idx_v, msg_v, acc_sh):
        c = lax.axis_index("c")
        s = lax.axis_index("s")
        wid = s * _NC + c
        pltpu.sync_copy(
            zero_hbm.at[pl.ds(s * rows_per_s, rows_per_s)],
            acc_sh.at[pl.ds(s * rows_per_s, rows_per_s)],
        )
        plsc.subcore_barrier()
        pltpu.sync_copy(idx_hbm.at[wid], idx_v)
        for hf in range(ngrp):
            pltpu.sync_copy(
                msg_hbm.at[pl.ds(wid * per_w + hf * grp, grp)], msg_v)
            for j in range(kg):
                pltpu.sync_copy(
                    msg_v.at[pl.ds(j * _CHUNK, _CHUNK)],
                    acc_sh.at[idx_v.at[hf * kg + j]],
                    add=True,
                )
        plsc.subcore_barrier()
        pltpu.sync_copy(
            acc_sh.at[pl.ds(s * rows_per_s, rows_per_s)],
            out_hbm.at[pl.ds(c * n_sc + s * rows_per_s, rows_per_s)],
        )

    fn = pl.kernel(
        body,
        out_type=jax.ShapeDtypeStruct((2 * n_sc, d), jnp.float32),
        mesh=mesh,
        scratch_types=[
            pltpu.VMEM((k, _CHUNK), jnp.int32),
            pltpu.VMEM((grp, d), jnp.float32),
            pltpu.VMEM_SHARED((n_sc, d), jnp.float32),
        ],
        compiler_params=pltpu.CompilerParams(use_tc_tiling_on_sc=False),
    )
    return fn(msg, idx3d, zeros_init)


def _edge_messages(ea_ext, xj, We1e, We2e, R128):
    e_pad, bfe = ea_ext.shape
    kc = We2e.shape[1]
    nblocks = kc // 128
    h_d = We1e.shape[1]

    def body(ea_ref, xj_ref, We1_ref, We2_ref, R_ref, out_ref):
        h = jnp.maximum(
            jnp.dot(ea_ref[...], We1_ref[...],
                    preferred_element_type=jnp.float32), 0.0)
        hb = h.astype(jnp.bfloat16)
        xb = xj_ref[...].astype(jnp.bfloat16)
        ngr = 4
        gw = kc // ngr
        acc = None
        for gr in range(ngr):
            gsl = slice(gr * gw, (gr + 1) * gw)
            Wt = jnp.dot(hb, We2_ref[:, gsl],
                         preferred_element_type=jnp.float32)
            XR = jnp.dot(xb, R_ref[:, gsl],
                         preferred_element_type=jnp.float32)
            for j in range(gw // 128):
                sl = slice(j * 128, (j + 1) * 128)
                t = Wt[:, sl] * XR[:, sl]
                acc = t if acc is None else acc + t
        out_ref[...] = acc

    return pl.pallas_call(
        body,
        grid=(e_pad // _EB,),
        in_specs=[
            pl.BlockSpec((_EB, bfe), lambda i: (i, 0)),
            pl.BlockSpec((_EB, 128), lambda i: (i, 0)),
            pl.BlockSpec((bfe, h_d), lambda i: (0, 0)),
            pl.BlockSpec((h_d, kc), lambda i: (0, 0)),
            pl.BlockSpec((128, kc), lambda i: (0, 0)),
        ],
        out_specs=pl.BlockSpec((_EB, 128), lambda i: (i, 0)),
        out_shape=jax.ShapeDtypeStruct((e_pad, 128), jnp.float32),
    )(ea_ext, xj, We1e, We2e, R128)


def _node_update(partsA, partsB, x, root, bias, g, b, n, n_sc):
    def body(pa_ref, pb_ref, x_ref, root_ref, bias_ref, g_ref, b_ref,
             out_ref):
        s128 = (pa_ref[0:n, :] + pa_ref[n_sc:n_sc + n, :]
                + pb_ref[0:n, :] + pb_ref[n_sc:n_sc + n, :])
        agg = (s128[:, 0:64] + s128[:, 64:128]
               + jnp.dot(x_ref[:, 0:64], root_ref[...],
                         preferred_element_type=jnp.float32)
               + bias_ref[...])
        m = jnp.mean(agg, axis=-1, keepdims=True)
        cen = agg - m
        v = jnp.mean(cen * cen, axis=-1, keepdims=True)
        hout = jnp.maximum(
            cen * lax.rsqrt(v + 1e-5) * g_ref[...] + b_ref[...], 0.0)
        out_ref[...] = jnp.concatenate(
            [hout, jnp.zeros_like(hout)], axis=1)

    return pl.pallas_call(
        body,
        out_shape=jax.ShapeDtypeStruct((n, 128), jnp.float32),
    )(partsA, partsB, x, root, bias, g, b)


def _pool_head(h, batch_col, fc1_W, fc1_b, fc2_W, fc2_b, gpad):
    n = h.shape[0]
    out_d = fc2_W.shape[1]

    def body(h_ref, batch_ref, w1_ref, b1_ref, w2_ref, b2_ref, out_ref):
        gids = lax.broadcasted_iota(jnp.int32, (1, gpad), 1)
        oh = (batch_ref[...] == gids).astype(jnp.float32)
        dn = (((0,), (0,)), ((), ()))
        sums = lax.dot_general(oh, h_ref[:, 0:64], dn,
                               preferred_element_type=jnp.float32)
        ones = jnp.ones((n, 1), jnp.float32)
        cnts = lax.dot_general(oh, ones, dn,
                               preferred_element_type=jnp.float32)
        hg = sums / jnp.maximum(cnts, 1.0)
        h2 = jnp.maximum(
            jnp.dot(hg, w1_ref[...], preferred_element_type=jnp.float32)
            + b1_ref[...], 0.0)
        out_ref[...] = (
            jnp.dot(h2, w2_ref[...], preferred_element_type=jnp.float32)
            + b2_ref[...])

    return pl.pallas_call(
        body,
        out_shape=jax.ShapeDtypeStruct((gpad, out_d), jnp.float32),
    )(h, batch_col, fc1_W, fc1_b, fc2_W, fc2_b)


def kernel(x, edge_index, edge_attr, batch,
           We1_1, be1_1, We2_1, be2_1, root1, bias1, g1, b1,
           We1_2, be1_2, We2_2, be2_2, root2, bias2, g2, b2,
           We1_3, be1_3, We2_3, be2_3, root3, bias3, g3, b3,
           fc1_W, fc1_b, fc2_W, fc2_b):
    n, nf = x.shape
    e = edge_index.shape[1]
    bf = edge_attr.shape[1]
    num_graphs = 50

    align = _NW * _CHUNK
    e_pad = -(-e // align) * align
    n_sc = -(-(n + 1) // (_NS * 8)) * (_NS * 8)

    pad_e = e_pad - e
    src = jnp.concatenate([edge_index[0], jnp.zeros((pad_e,), jnp.int32)])
    dump = n + (jnp.arange(pad_e, dtype=jnp.int32) % (n_sc - n))
    dst = jnp.concatenate([edge_index[1], dump])
    e_half = e_pad // 2
    kk = e_half // (_NW * _CHUNK)
    src0 = src[:e_half].reshape(_NW, kk, _CHUNK)
    src1 = src[e_half:].reshape(_NW, kk, _CHUNK)
    dst0 = dst[:e_half].reshape(_NW, kk, _CHUNK)
    dst1 = dst[e_half:].reshape(_NW, kk, _CHUNK)
    ea_ext = jnp.concatenate(
        [edge_attr, jnp.ones((e, 1), jnp.float32)], axis=1)
    ea_ext = jnp.concatenate(
        [ea_ext, jnp.zeros((pad_e, bf + 1), jnp.float32)], axis=0)
    ea0 = ea_ext[:e_pad // 2]
    ea1 = ea_ext[e_pad // 2:]
    zeros_init = jnp.zeros((n_sc, 128), jnp.float32)
    R = jnp.kron(jnp.eye(64, dtype=jnp.float32),
                 jnp.ones((1, 64), jnp.float32))
    R128 = jnp.concatenate(
        [R, jnp.zeros((64, R.shape[1]), jnp.float32)],
        axis=0).astype(jnp.bfloat16)

    layers = [
        (We1_1, be1_1, We2_1, be2_1, root1, bias1, g1, b1),
        (We1_2, be1_2, We2_2, be2_2, root2, bias2, g2, b2),
        (We1_3, be1_3, We2_3, be2_3, root3, bias3, g3, b3),
    ]

    h = jnp.concatenate([x, jnp.zeros((n, 64), jnp.float32)], axis=1)
    for We1, be1, We2, be2, root, bias, g, b in layers:
        top = jnp.concatenate([We1, jnp.zeros((bf, 1), jnp.float32)], axis=1)
        bot = jnp.concatenate(
            [be1[None, :], jnp.ones((1, 1), jnp.float32)], axis=1)
        We1e = jnp.concatenate([top, bot], axis=0)
        We2e = jnp.concatenate(
            [We2, be2[None, :]], axis=0).astype(jnp.bfloat16)

        xj0 = _sc_gather(h, src0, e_half)
        msg0 = _edge_messages(ea0, xj0, We1e, We2e, R128)
        xj1 = _sc_gather(h, src1, e_half)
        partsA = _sc_scatter_add(msg0, dst0, zeros_init, n_sc)
        msg1 = _edge_messages(ea1, xj1, We1e, We2e, R128)
        partsB = _sc_scatter_add(msg1, dst1, zeros_init, n_sc)
        h = _node_update(partsA, partsB, h, root, bias.reshape(1, 64),
                         g.reshape(1, 64), b.reshape(1, 64), n, n_sc)

    out = _pool_head(h, batch.reshape(n, 1).astype(jnp.int32),
                     fc1_W, fc1_b.reshape(1, -1), fc2_W, fc2_b.reshape(1, -1),
                     gpad=64)
    return out[:num_graphs]
